# baseline (device time: 8463 ns/iter reference)
import jax
import jax.numpy as jnp
from jax import lax
from jax.experimental import pallas as pl
from jax.experimental.pallas import tpu as pltpu

N_GLOBAL = 2048
G = 4


def kernel(x):
    m_per, n_per = x.shape
    bm = m_per // G
    rows = bm // 128
    inv = 1.0 / N_GLOBAL

    def body(x_ref, out_ref, send_buf, recv_buf, send_sem, recv_sem):
        i = pl.program_id(0)
        my_x = lax.axis_index("x")
        my_y = lax.axis_index("y")
        nbr = (my_x, 1 - my_y)

        barrier_sem = pltpu.get_barrier_semaphore()

        p = jnp.sum(x_ref[:, :], axis=1)
        send_buf[pl.ds(i * rows, rows), :] = p.reshape(rows, 128)

        @pl.when(i == 0)
        def _():
            pl.semaphore_signal(
                barrier_sem, inc=1, device_id=nbr,
                device_id_type=pl.DeviceIdType.MESH,
            )

        @pl.when(i == G - 1)
        def _():
            pl.semaphore_wait(barrier_sem, 1)
            rdma = pltpu.make_async_remote_copy(
                src_ref=send_buf,
                dst_ref=recv_buf,
                send_sem=send_sem,
                recv_sem=recv_sem,
                device_id=nbr,
                device_id_type=pl.DeviceIdType.MESH,
            )
            rdma.start()
            rdma.wait()
            out_ref[:, :] = (send_buf[:, :] + recv_buf[:, :]) * inv

    res = pl.pallas_call(
        body,
        grid=(G,),
        out_shape=jax.ShapeDtypeStruct((G * rows, 128), jnp.float32),
        in_specs=[
            pl.BlockSpec((bm, n_per), lambda i: (i, 0),
                         memory_space=pltpu.VMEM),
        ],
        out_specs=pl.BlockSpec((G * rows, 128), lambda i: (0, 0),
                               memory_space=pltpu.VMEM),
        scratch_shapes=[
            pltpu.VMEM((G * rows, 128), jnp.float32),
            pltpu.VMEM((G * rows, 128), jnp.float32),
            pltpu.SemaphoreType.DMA,
            pltpu.SemaphoreType.DMA,
        ],
        compiler_params=pltpu.CompilerParams(collective_id=0),
    )(x)
    return jnp.reshape(res, (m_per, 1))
